# baseline (speedup 1.0000x reference)
import functools

import jax
import jax.numpy as jnp
from jax import lax
from jax.experimental import pallas as pl
from jax.experimental.pallas import tpu as pltpu
from jax.experimental.pallas import tpu_sc as plsc

BATCH = 16384
EMBED_DIM = 64
LANES = 16
NUM_CORES = 2
NUM_SUBCORES = 16
NUM_WORKERS = NUM_CORES * NUM_SUBCORES
BPW = BATCH // NUM_WORKERS
CHUNK = 128
NCHUNKS = BPW // CHUNK


def _dot_body(i_hbm, j_hbm, emb_hbm, out_hbm,
              idx_i, idx_j, rows_i, rows_j, out_v, sem_i, sem_j):
    c = lax.axis_index("c")
    s = lax.axis_index("s")
    wid = s * NUM_CORES + c
    base = wid * BPW

    pltpu.sync_copy(i_hbm.at[wid], idx_i)
    pltpu.sync_copy(j_hbm.at[wid], idx_j)

    copies = []
    for k in range(NCHUNKS):
        copies.append(pltpu.async_copy(
            emb_hbm.at[idx_i.at[k]], rows_i.at[pl.ds(k * CHUNK, CHUNK)], sem_i))
        copies.append(pltpu.async_copy(
            emb_hbm.at[idx_j.at[k]], rows_j.at[pl.ds(k * CHUNK, CHUNK)], sem_j))
    for cp in copies:
        cp.wait()

    lane = lax.broadcasted_iota(jnp.int32, (LANES,), 0)

    def group(g, _):
        out_vec = jnp.zeros((LANES,), jnp.float32)
        for rl in range(LANES):
            r = g * LANES + rl
            acc = jnp.zeros((LANES,), jnp.float32)
            for d in range(EMBED_DIM // LANES):
                vi = rows_i[r, pl.ds(d * LANES, LANES)]
                vj = rows_j[r, pl.ds(d * LANES, LANES)]
                acc = acc + vi * vj
            dot = jnp.sum(acc)
            out_vec = jnp.where(lane == rl, dot, out_vec)
        out_v[pl.ds(g * LANES, LANES)] = out_vec
        return 0

    lax.fori_loop(0, BPW // LANES, group, 0)

    pltpu.sync_copy(out_v, out_hbm.at[pl.ds(base, BPW)])


@functools.partial(jax.jit, donate_argnums=())
def _sc_dot(i, j, node_emb):
    mesh = plsc.VectorSubcoreMesh(core_axis_name="c", subcore_axis_name="s")
    kfn = pl.kernel(
        _dot_body,
        mesh=mesh,
        compiler_params=pltpu.CompilerParams(
            needs_layout_passes=False, use_tc_tiling_on_sc=False),
        out_type=jax.ShapeDtypeStruct((BATCH,), jnp.float32),
        scratch_types=[
            pltpu.VMEM((NCHUNKS, CHUNK), jnp.int32),
            pltpu.VMEM((NCHUNKS, CHUNK), jnp.int32),
            pltpu.VMEM((BPW, EMBED_DIM), jnp.float32),
            pltpu.VMEM((BPW, EMBED_DIM), jnp.float32),
            pltpu.VMEM((BPW,), jnp.float32),
            pltpu.SemaphoreType.DMA,
            pltpu.SemaphoreType.DMA,
        ],
    )
    return kfn(i.reshape(NUM_WORKERS, NCHUNKS, CHUNK),
               j.reshape(NUM_WORKERS, NCHUNKS, CHUNK), node_emb)


def kernel(i, j, node_emb):
    return _sc_dot(i.astype(jnp.int32), j.astype(jnp.int32), node_emb)
